# initial kernel scaffold (unmeasured)
import jax
import jax.numpy as jnp
from jax import lax
from jax.experimental import pallas as pl
from jax.experimental.pallas import tpu as pltpu


def _matmul_body(a_ref, b_ref, o_ref):
    o_ref[...] = jnp.dot(a_ref[...], b_ref[...],
                         preferred_element_type=jnp.float32)


def _matmul(a, b, bm=512):
    m, k = a.shape
    _, n = b.shape
    return pl.pallas_call(
        _matmul_body,
        grid=(m // bm,),
        in_specs=[
            pl.BlockSpec((bm, k), lambda i: (i, 0)),
            pl.BlockSpec((k, n), lambda i: (0, 0)),
        ],
        out_specs=pl.BlockSpec((bm, n), lambda i: (i, 0)),
        out_shape=jax.ShapeDtypeStruct((m, n), jnp.float32),
        compiler_params=pltpu.CompilerParams(
            dimension_semantics=("arbitrary",),
        ),
    )(a, b)


def _exchange_body(p_ref, o_ref, send_sem, recv_sem):
    my_x = lax.axis_index("x")
    my_y = lax.axis_index("y")
    my_z = lax.axis_index("z")
    nbr = (my_x, 1 - my_y, my_z)

    barrier = pltpu.get_barrier_semaphore()
    pl.semaphore_signal(barrier, inc=1, device_id=nbr,
                        device_id_type=pl.DeviceIdType.MESH)
    pl.semaphore_wait(barrier, 1)

    sh = o_ref.shape[1]
    rdma = pltpu.make_async_remote_copy(
        src_ref=p_ref.at[:, pl.ds((1 - my_y) * sh, sh), :],
        dst_ref=o_ref,
        send_sem=send_sem,
        recv_sem=recv_sem,
        device_id=nbr,
        device_id_type=pl.DeviceIdType.MESH,
    )
    rdma.start()
    rdma.wait_send()
    rdma.wait_recv()
    o_ref[...] = o_ref[...] + p_ref[:, pl.ds(my_y * sh, sh), :]


def _exchange_add(p):
    b, s, n = p.shape
    return pl.pallas_call(
        _exchange_body,
        out_shape=jax.ShapeDtypeStruct((b, s // 2, n), jnp.float32),
        in_specs=[pl.BlockSpec(memory_space=pltpu.VMEM)],
        out_specs=pl.BlockSpec(memory_space=pltpu.VMEM),
        scratch_shapes=[pltpu.SemaphoreType.DMA, pltpu.SemaphoreType.DMA],
        compiler_params=pltpu.CompilerParams(collective_id=0),
    )(p)


def kernel(O, Wo):
    B, S, Hl, D = O.shape
    K = Hl * D
    N = Wo.shape[1]
    A = O.reshape(B * S, K)
    P = _matmul(A, Wo).reshape(B, S, N)
    return _exchange_add(P)


# baseline (device time: 527662 ns/iter reference)
import jax
import jax.numpy as jnp
from jax import lax
from jax.experimental import pallas as pl
from jax.experimental.pallas import tpu as pltpu


def _matmul_body(a_ref, b_ref, o_ref):
    o_ref[...] = jnp.dot(a_ref[...], b_ref[...],
                         preferred_element_type=jnp.float32)


def _matmul(a, b, bm=512):
    m, k = a.shape
    _, n = b.shape
    return pl.pallas_call(
        _matmul_body,
        grid=(m // bm,),
        in_specs=[
            pl.BlockSpec((bm, k), lambda i: (i, 0)),
            pl.BlockSpec((k, n), lambda i: (0, 0)),
        ],
        out_specs=pl.BlockSpec((bm, n), lambda i: (i, 0)),
        out_shape=jax.ShapeDtypeStruct((m, n), jnp.float32),
        compiler_params=pltpu.CompilerParams(
            dimension_semantics=("arbitrary",),
            vmem_limit_bytes=60 * 1024 * 1024,
        ),
    )(a, b)


def _exchange_body(p_ref, o_ref, buf, copy_sems, send_sems, recv_sems):
    nb = o_ref.shape[0]
    my_x = lax.axis_index("x")
    my_y = lax.axis_index("y")
    my_z = lax.axis_index("z")
    nbr = (my_x, 1 - my_y, my_z)

    barrier = pltpu.get_barrier_semaphore()
    pl.semaphore_signal(barrier, inc=1, device_id=nbr,
                        device_id_type=pl.DeviceIdType.MESH)
    pl.semaphore_wait(barrier, 1)

    sh = o_ref.shape[1]
    mine = my_y * sh
    other = (1 - my_y) * sh
    for b in range(nb):
        slot = b % 2
        cp = pltpu.make_async_copy(
            p_ref.at[b, pl.ds(mine, sh), :], buf.at[slot], copy_sems.at[slot]
        )
        cp.start()
        rdma = pltpu.make_async_remote_copy(
            src_ref=p_ref.at[b, pl.ds(other, sh), :],
            dst_ref=o_ref.at[b],
            send_sem=send_sems.at[b],
            recv_sem=recv_sems.at[b],
            device_id=nbr,
            device_id_type=pl.DeviceIdType.MESH,
        )
        rdma.start()
        cp.wait()
        rdma.wait_send()
        rdma.wait_recv()
        o_ref[b] = o_ref[b] + buf[slot]


def _exchange_add(p):
    b, s, n = p.shape
    return pl.pallas_call(
        _exchange_body,
        out_shape=jax.ShapeDtypeStruct((b, s // 2, n), jnp.float32),
        in_specs=[pl.BlockSpec(memory_space=pl.ANY)],
        out_specs=pl.BlockSpec(memory_space=pltpu.VMEM),
        scratch_shapes=[
            pltpu.VMEM((2, s // 2, n), jnp.float32),
            pltpu.SemaphoreType.DMA((2,)),
            pltpu.SemaphoreType.DMA((b,)),
            pltpu.SemaphoreType.DMA((b,)),
        ],
        compiler_params=pltpu.CompilerParams(
            collective_id=0,
            vmem_limit_bytes=60 * 1024 * 1024,
        ),
    )(p)


def kernel(O, Wo):
    B, S, Hl, D = O.shape
    K = Hl * D
    N = Wo.shape[1]
    A = O.reshape(B * S, K)
    P = _matmul(A, Wo).reshape(B, S, N)
    return _exchange_add(P)


# device time: 457287 ns/iter; 1.1539x vs baseline; 1.1539x over previous
import jax
import jax.numpy as jnp
from jax import lax
from jax.experimental import pallas as pl
from jax.experimental.pallas import tpu as pltpu

_BM = 256
_BN = 2048
_NCH = 2
_NT = 2


def _fused_body(a_ref, wo_ref, o_ref, wo_buf, a_buf, send_buf,
                wo_sem, a_sem, send_sems, recv_sems):
    nb, sh, n = o_ref.shape
    k = a_ref.shape[1]
    my_x = lax.axis_index("x")
    my_y = lax.axis_index("y")
    my_z = lax.axis_index("z")
    nbr = (my_x, 1 - my_y, my_z)
    s_full = sh * 2

    barrier = pltpu.get_barrier_semaphore()
    pl.semaphore_signal(barrier, inc=1, device_id=nbr,
                        device_id_type=pl.DeviceIdType.MESH)
    pl.semaphore_wait(barrier, 1)

    mine = my_y * sh
    other = (1 - my_y) * sh

    rdmas = []

    def load_a(b, q, half_off, slot):
        row = b * s_full + half_off + q * _BM
        cp = pltpu.make_async_copy(
            a_ref.at[pl.ds(row, _BM), :], a_buf.at[slot], a_sem
        )
        cp.start()
        cp.wait()

    for ch in range(_NCH):
        wo_cp = pltpu.make_async_copy(
            wo_ref.at[:, pl.ds(ch * _BN, _BN)], wo_buf, wo_sem
        )
        wo_cp.start()
        wo_cp.wait()

        for t in range(nb * _NT):
            b, q = t // _NT, t % _NT
            slot = t % 2
            kidx = ch * nb * _NT + t
            load_a(b, q, other, slot)
            if kidx >= 2:
                rdmas[kidx - 2].wait_send()
            send_buf[slot] = jnp.dot(a_buf[slot], wo_buf[...],
                                     preferred_element_type=jnp.float32)
            rdma = pltpu.make_async_remote_copy(
                src_ref=send_buf.at[slot],
                dst_ref=o_ref.at[b, pl.ds(q * _BM, _BM),
                                 pl.ds(ch * _BN, _BN)],
                send_sem=send_sems.at[kidx],
                recv_sem=recv_sems.at[kidx],
                device_id=nbr,
                device_id_type=pl.DeviceIdType.MESH,
            )
            rdma.start()
            rdmas.append(rdma)

        for t in range(nb * _NT):
            b, q = t // _NT, t % _NT
            slot = t % 2
            kidx = ch * nb * _NT + t
            load_a(b, q, mine, slot)
            v = jnp.dot(a_buf[slot], wo_buf[...],
                        preferred_element_type=jnp.float32)
            rdmas[kidx].wait_recv()
            o_ref[b, pl.ds(q * _BM, _BM), pl.ds(ch * _BN, _BN)] = (
                o_ref[b, pl.ds(q * _BM, _BM), pl.ds(ch * _BN, _BN)] + v
            )

    rdmas[-2].wait_send()
    rdmas[-1].wait_send()


def kernel(O, Wo):
    B, S, Hl, D = O.shape
    K = Hl * D
    N = Wo.shape[1]
    A = O.reshape(B * S, K)
    n_slots = _NCH * B * _NT
    return pl.pallas_call(
        _fused_body,
        out_shape=jax.ShapeDtypeStruct((B, S // 2, N), jnp.float32),
        in_specs=[
            pl.BlockSpec(memory_space=pl.ANY),
            pl.BlockSpec(memory_space=pl.ANY),
        ],
        out_specs=pl.BlockSpec(memory_space=pltpu.VMEM),
        scratch_shapes=[
            pltpu.VMEM((K, _BN), jnp.float32),
            pltpu.VMEM((2, _BM, K), jnp.float32),
            pltpu.VMEM((2, _BM, _BN), jnp.float32),
            pltpu.SemaphoreType.DMA,
            pltpu.SemaphoreType.DMA,
            pltpu.SemaphoreType.DMA((n_slots,)),
            pltpu.SemaphoreType.DMA((n_slots,)),
        ],
        compiler_params=pltpu.CompilerParams(
            collective_id=0,
            vmem_limit_bytes=62 * 1024 * 1024,
        ),
    )(A, Wo)


# device time: 447871 ns/iter; 1.1782x vs baseline; 1.0210x over previous
import jax
import jax.numpy as jnp
from jax import lax
from jax.experimental import pallas as pl
from jax.experimental.pallas import tpu as pltpu

_BM = 256
_BN = 2048
_NCH = 2
_NT = 2
_NH = 16
_HD = 128


def _fused_body(o_in, wo_ref, o_ref, wo_buf, a_buf, send_buf,
                wo_sem, a_sem, send_sems, recv_sems):
    nb = o_ref.shape[0]
    my_x = lax.axis_index("x")
    my_y = lax.axis_index("y")
    my_z = lax.axis_index("z")
    nbr = (my_x, 1 - my_y, my_z)

    barrier = pltpu.get_barrier_semaphore()
    pl.semaphore_signal(barrier, inc=1, device_id=nbr,
                        device_id_type=pl.DeviceIdType.MESH)
    pl.semaphore_wait(barrier, 1)

    sh = _NT * _BM
    mine = my_y * sh
    other = (1 - my_y) * sh
    npt = nb * _NT

    def load_a(b, row, slot):
        for h in range(_NH):
            pltpu.make_async_copy(
                o_in.at[b, pl.ds(row, _BM), h, :],
                a_buf.at[slot, h],
                a_sem,
            ).start()
        for h in range(_NH):
            pltpu.make_async_copy(
                o_in.at[b, pl.ds(row, _BM), h, :],
                a_buf.at[slot, h],
                a_sem,
            ).wait()

    def tile_dot(slot):
        v = jnp.dot(a_buf[slot, 0], wo_buf[0:_HD, :],
                    preferred_element_type=jnp.float32)
        for h in range(1, _NH):
            v = v + jnp.dot(a_buf[slot, h],
                            wo_buf[h * _HD:(h + 1) * _HD, :],
                            preferred_element_type=jnp.float32)
        return v

    def mk_rdma(b, q, ch, kidx, slot):
        return pltpu.make_async_remote_copy(
            src_ref=send_buf.at[slot],
            dst_ref=o_ref.at[b, q, :, pl.ds(ch * _BN, _BN)],
            send_sem=send_sems.at[kidx],
            recv_sem=recv_sems.at[kidx],
            device_id=nbr,
            device_id_type=pl.DeviceIdType.MESH,
        )

    for ch in range(_NCH):
        wo_cp = pltpu.make_async_copy(
            wo_ref.at[:, pl.ds(ch * _BN, _BN)], wo_buf, wo_sem
        )
        wo_cp.start()
        wo_cp.wait()

        def p1_body(t, carry):
            b = t // _NT
            q = lax.rem(t, _NT)
            slot = lax.rem(t, 2)
            kidx = ch * npt + t
            @pl.when(kidx >= 2)
            def _():
                mk_rdma(b, q, ch, kidx - 2, slot).wait_send()
            load_a(b, other + q * _BM, slot)
            send_buf[slot] = tile_dot(slot)
            mk_rdma(b, q, ch, kidx, slot).start()
            return carry

        lax.fori_loop(0, npt, p1_body, 0)

        def p2_body(t, carry):
            b = t // _NT
            q = lax.rem(t, _NT)
            slot = lax.rem(t, 2)
            kidx = ch * npt + t
            load_a(b, mine + q * _BM, slot)
            v = tile_dot(slot)
            mk_rdma(b, q, ch, kidx, slot).wait_recv()
            o_ref[b, q, :, pl.ds(ch * _BN, _BN)] = (
                o_ref[b, q, :, pl.ds(ch * _BN, _BN)] + v
            )
            return carry

        lax.fori_loop(0, npt, p2_body, 0)

    last = _NCH * npt
    mk_rdma(nb - 1, _NT - 2, _NCH - 1, last - 2, 0).wait_send()
    mk_rdma(nb - 1, _NT - 1, _NCH - 1, last - 1, 1).wait_send()


def kernel(O, Wo):
    B, S, Hl, D = O.shape
    N = Wo.shape[1]
    n_slots = _NCH * B * _NT
    out = pl.pallas_call(
        _fused_body,
        out_shape=jax.ShapeDtypeStruct((B, _NT, _BM, N), jnp.float32),
        in_specs=[
            pl.BlockSpec(memory_space=pl.ANY),
            pl.BlockSpec(memory_space=pl.ANY),
        ],
        out_specs=pl.BlockSpec(memory_space=pltpu.VMEM),
        scratch_shapes=[
            pltpu.VMEM((Hl * D, _BN), jnp.float32),
            pltpu.VMEM((2, _NH, _BM, _HD), jnp.float32),
            pltpu.VMEM((2, _BM, _BN), jnp.float32),
            pltpu.SemaphoreType.DMA,
            pltpu.SemaphoreType.DMA,
            pltpu.SemaphoreType.DMA((n_slots,)),
            pltpu.SemaphoreType.DMA((n_slots,)),
        ],
        compiler_params=pltpu.CompilerParams(
            collective_id=0,
            vmem_limit_bytes=62 * 1024 * 1024,
        ),
    )(O, Wo)
    return out.reshape(B, _NT * _BM, N)


# device time: 421057 ns/iter; 1.2532x vs baseline; 1.0637x over previous
import jax
import jax.numpy as jnp
from jax import lax
from jax.experimental import pallas as pl
from jax.experimental.pallas import tpu as pltpu

_BM = 256
_BN = 2048
_NCH = 2
_NT = 2
_NH = 16
_HD = 128


def _fused_body(o_in, wo_ref, o_ref, wo_buf, a_buf, send_buf,
                wo_sem, a_sems, send_sems, recv_sems):
    nb = o_ref.shape[0]
    my_x = lax.axis_index("x")
    my_y = lax.axis_index("y")
    my_z = lax.axis_index("z")
    nbr = (my_x, 1 - my_y, my_z)

    barrier = pltpu.get_barrier_semaphore()
    pl.semaphore_signal(barrier, inc=1, device_id=nbr,
                        device_id_type=pl.DeviceIdType.MESH)
    pl.semaphore_wait(barrier, 1)

    sh = _NT * _BM
    mine = my_y * sh
    other = (1 - my_y) * sh
    npt = nb * _NT

    def load_start(b, row, slot):
        for h in range(_NH):
            pltpu.make_async_copy(
                o_in.at[b, pl.ds(row, _BM), h, :],
                a_buf.at[slot, :, pl.ds(h * _HD, _HD)],
                a_sems.at[slot],
            ).start()

    def load_wait(slot):
        for h in range(_NH):
            pltpu.make_async_copy(
                o_in.at[0, pl.ds(0, _BM), h, :],
                a_buf.at[slot, :, pl.ds(h * _HD, _HD)],
                a_sems.at[slot],
            ).wait()

    def mk_rdma(b, q, ch, kidx, slot):
        return pltpu.make_async_remote_copy(
            src_ref=send_buf.at[slot],
            dst_ref=o_ref.at[b, q, :, pl.ds(ch * _BN, _BN)],
            send_sem=send_sems.at[kidx],
            recv_sem=recv_sems.at[kidx],
            device_id=nbr,
            device_id_type=pl.DeviceIdType.MESH,
        )

    for ch in range(_NCH):
        wo_cp = pltpu.make_async_copy(
            wo_ref.at[:, pl.ds(ch * _BN, _BN)], wo_buf, wo_sem
        )
        wo_cp.start()
        wo_cp.wait()

        load_start(0, other, 0)

        def p1_body(t, carry):
            b = t // _NT
            q = lax.rem(t, _NT)
            slot = lax.rem(t, 2)
            kidx = ch * npt + t

            @pl.when(t + 1 < npt)
            def _():
                bn = (t + 1) // _NT
                qn = lax.rem(t + 1, _NT)
                load_start(bn, other + qn * _BM, lax.rem(t + 1, 2))

            @pl.when(kidx >= 2)
            def _():
                mk_rdma(b, q, ch, kidx - 2, slot).wait_send()

            load_wait(slot)
            send_buf[slot] = jnp.dot(a_buf[slot], wo_buf[...],
                                     preferred_element_type=jnp.float32)
            mk_rdma(b, q, ch, kidx, slot).start()
            return carry

        lax.fori_loop(0, npt, p1_body, 0)

        load_start(0, mine, 0)

        def p2_body(t, carry):
            b = t // _NT
            q = lax.rem(t, _NT)
            slot = lax.rem(t, 2)
            kidx = ch * npt + t

            @pl.when(t + 1 < npt)
            def _():
                bn = (t + 1) // _NT
                qn = lax.rem(t + 1, _NT)
                load_start(bn, mine + qn * _BM, lax.rem(t + 1, 2))

            load_wait(slot)
            v = jnp.dot(a_buf[slot], wo_buf[...],
                        preferred_element_type=jnp.float32)
            mk_rdma(b, q, ch, kidx, slot).wait_recv()
            o_ref[b, q, :, pl.ds(ch * _BN, _BN)] = (
                o_ref[b, q, :, pl.ds(ch * _BN, _BN)] + v
            )
            return carry

        lax.fori_loop(0, npt, p2_body, 0)

    last = _NCH * npt
    mk_rdma(nb - 1, 0, _NCH - 1, last - 2, 0).wait_send()
    mk_rdma(nb - 1, 1, _NCH - 1, last - 1, 1).wait_send()


def kernel(O, Wo):
    B, S, Hl, D = O.shape
    N = Wo.shape[1]
    n_slots = _NCH * B * _NT
    out = pl.pallas_call(
        _fused_body,
        out_shape=jax.ShapeDtypeStruct((B, _NT, _BM, N), jnp.float32),
        in_specs=[
            pl.BlockSpec(memory_space=pl.ANY),
            pl.BlockSpec(memory_space=pl.ANY),
        ],
        out_specs=pl.BlockSpec(memory_space=pltpu.VMEM),
        scratch_shapes=[
            pltpu.VMEM((Hl * D, _BN), jnp.float32),
            pltpu.VMEM((2, _BM, Hl * D), jnp.float32),
            pltpu.VMEM((2, _BM, _BN), jnp.float32),
            pltpu.SemaphoreType.DMA,
            pltpu.SemaphoreType.DMA((2,)),
            pltpu.SemaphoreType.DMA((n_slots,)),
            pltpu.SemaphoreType.DMA((n_slots,)),
        ],
        compiler_params=pltpu.CompilerParams(
            collective_id=0,
            vmem_limit_bytes=62 * 1024 * 1024,
        ),
    )(O, Wo)
    return out.reshape(B, _NT * _BM, N)
